# baseline (speedup 1.0000x reference)
import functools

import numpy as np

import jax
import jax.numpy as jnp
from jax.experimental import pallas as pl
from jax.experimental.pallas import tpu as pltpu

EPS = 1e-5


def _interp_matrix(d_in, d_out):
    if d_out == 1:
        src = np.zeros((1,), np.float64)
    else:
        src = np.arange(d_out, dtype=np.float64) * ((d_in - 1) / (d_out - 1))
    lo = np.clip(np.floor(src).astype(np.int64), 0, d_in - 1)
    hi = np.minimum(lo + 1, d_in - 1)
    w_hi = (src - lo).astype(np.float32)
    m = np.zeros((d_out, d_in), np.float32)
    m[np.arange(d_out), lo] += 1.0 - w_hi
    m[np.arange(d_out), hi] += w_hi
    return jnp.asarray(m)


def _conv_stats_kernel(x3_ref, w_ref, y_ref, st_ref, *, H, W, C):
    hp = H + 2
    acc = None
    for dx in range(3):
        for dy in range(3):
            base = dx * hp + dy
            xs = x3_ref[0, base:base + H, :, :].reshape(H * W, C)
            p = jnp.dot(xs, w_ref[3 * dy + dx],
                        preferred_element_type=jnp.float32)
            acc = p if acc is None else acc + p
    y_ref[0] = acc.reshape(H, W, C).astype(y_ref.dtype)
    s0 = jnp.sum(acc, axis=0, keepdims=True)
    s1 = jnp.sum(acc * acc, axis=0, keepdims=True)
    st_ref[0] = jnp.concatenate([s0, s1], axis=0)


def _conv_stats(x3_bf16, w9_bf16):
    n, hp3, w, c = x3_bf16.shape
    hp = hp3 // 3
    h = hp - 2
    return pl.pallas_call(
        functools.partial(_conv_stats_kernel, H=h, W=w, C=c),
        out_shape=[
            jax.ShapeDtypeStruct((n, h, w, c), jnp.bfloat16),
            jax.ShapeDtypeStruct((n, 2, c), jnp.float32),
        ],
        grid_spec=pltpu.PrefetchScalarGridSpec(
            num_scalar_prefetch=0,
            grid=(n,),
            in_specs=[
                pl.BlockSpec((1, hp3, w, c), lambda i: (i, 0, 0, 0)),
                pl.BlockSpec((9, c, c), lambda i: (0, 0, 0)),
            ],
            out_specs=[
                pl.BlockSpec((1, h, w, c), lambda i: (i, 0, 0, 0)),
                pl.BlockSpec((1, 2, c), lambda i: (i, 0, 0)),
            ],
        ),
        compiler_params=pltpu.CompilerParams(
            dimension_semantics=("arbitrary",)),
    )(x3_bf16, w9_bf16)


def _norm_resize_add_kernel(y_ref, st_ref, gb_ref, rh_ref, rwt_ref, res_ref,
                            o_ref, *, inv_m, Hs, Ws, Hd, Wd, C):
    s = jnp.sum(st_ref[...], axis=0)
    mean = s[0:1] * inv_m
    var = jnp.maximum(s[1:2] * inv_m - mean * mean, 0.0)
    scale = gb_ref[0:1] * jax.lax.rsqrt(var + EPS)
    shift = gb_ref[1:2] - mean * scale

    z = jnp.maximum(y_ref[0].astype(jnp.float32) * scale + shift, 0.0)
    t = jnp.dot(rh_ref[...], z.reshape(Hs, Ws * C),
                preferred_element_type=jnp.float32)
    tt = jnp.swapaxes(t.reshape(Hd, Ws, C), 1, 2)
    u = jnp.dot(tt.reshape(Hd * C, Ws), rwt_ref[...],
                preferred_element_type=jnp.float32)
    res = jnp.swapaxes(res_ref[0, 1:Hd + 1, 1:Wd + 1, :], 1, 2)
    o_ref[0] = (u.reshape(Hd, C, Wd)
                + res.astype(jnp.float32)).astype(o_ref.dtype)


def _norm_resize_add(y_raw, stats, gb, rh, rwt, res_pad):
    n, hs, ws, c = y_raw.shape
    hd, wd = rh.shape[0], rwt.shape[1]
    return pl.pallas_call(
        functools.partial(_norm_resize_add_kernel,
                          inv_m=1.0 / float(n * hs * ws),
                          Hs=hs, Ws=ws, Hd=hd, Wd=wd, C=c),
        out_shape=jax.ShapeDtypeStruct((n, hd, c, wd), jnp.bfloat16),
        grid_spec=pltpu.PrefetchScalarGridSpec(
            num_scalar_prefetch=0,
            grid=(n,),
            in_specs=[
                pl.BlockSpec((1, hs, ws, c), lambda i: (i, 0, 0, 0)),
                pl.BlockSpec((n, 2, c), lambda i: (0, 0, 0)),
                pl.BlockSpec((2, c), lambda i: (0, 0)),
                pl.BlockSpec((hd, hs), lambda i: (0, 0)),
                pl.BlockSpec((ws, wd), lambda i: (0, 0)),
                pl.BlockSpec((1, hd + 2, wd + 2, c), lambda i: (i, 0, 0, 0)),
            ],
            out_specs=pl.BlockSpec((1, hd, c, wd), lambda i: (i, 0, 0, 0)),
        ),
        compiler_params=pltpu.CompilerParams(
            dimension_semantics=("arbitrary",)),
    )(y_raw, stats, gb, rh, rwt, res_pad)


def _prep(x_nchw):
    xp = jnp.pad(jnp.transpose(x_nchw, (0, 2, 3, 1)),
                 ((0, 0), (1, 1), (1, 1), (0, 0))).astype(jnp.bfloat16)
    w = x_nchw.shape[3]
    x3 = jnp.concatenate([xp[:, :, 0:w, :], xp[:, :, 1:w + 1, :],
                          xp[:, :, 2:w + 2, :]], axis=1)
    return xp, x3


def kernel(x1, x2, w, b, gamma, beta):
    del b
    c = x1.shape[1]
    h1, w1 = x1.shape[2], x1.shape[3]
    h2, w2 = x2.shape[2], x2.shape[3]

    x1p, x1s = _prep(x1)
    x2p, x2s = _prep(x2)

    w9 = jnp.transpose(w, (2, 3, 1, 0)).reshape(9, c, c).astype(jnp.bfloat16)
    gb = jnp.stack([gamma, beta], axis=0)

    yA, stA = _conv_stats(x2s, w9)
    yB, stB = _conv_stats(x1s, w9)

    outA = _norm_resize_add(yA, stA, gb, _interp_matrix(h2, h1),
                            _interp_matrix(w2, w1).T, x1p)
    outB = _norm_resize_add(yB, stB, gb, _interp_matrix(h1, h2),
                            _interp_matrix(w1, w2).T, x2p)

    x_out = jnp.transpose(outA, (0, 2, 1, 3)).astype(jnp.float32)
    y_out = jnp.transpose(outB, (0, 2, 1, 3)).astype(jnp.float32)
    return x_out, y_out


# speedup vs baseline: 1.0471x; 1.0471x over previous
import functools

import numpy as np

import jax
import jax.numpy as jnp
from jax.experimental import pallas as pl
from jax.experimental.pallas import tpu as pltpu

EPS = 1e-5


def _interp_matrix(d_in, d_out):
    if d_out == 1:
        src = np.zeros((1,), np.float64)
    else:
        src = np.arange(d_out, dtype=np.float64) * ((d_in - 1) / (d_out - 1))
    lo = np.clip(np.floor(src).astype(np.int64), 0, d_in - 1)
    hi = np.minimum(lo + 1, d_in - 1)
    w_hi = (src - lo).astype(np.float32)
    m = np.zeros((d_out, d_in), np.float32)
    m[np.arange(d_out), lo] += 1.0 - w_hi
    m[np.arange(d_out), hi] += w_hi
    return jnp.asarray(m)


def _conv_stats_kernel(xp_ref, w_ref, y_ref, st_ref, *, H, W, C):
    acc = None
    for dy in range(3):
        for dx in range(3):
            xs = xp_ref[0, dy:dy + H, dx:dx + W, :].reshape(H * W, C)
            p = jnp.dot(xs, w_ref[3 * dy + dx],
                        preferred_element_type=jnp.float32)
            acc = p if acc is None else acc + p
    y_ref[0] = acc.reshape(H, W, C).astype(y_ref.dtype)
    s0 = jnp.sum(acc, axis=0, keepdims=True)
    s1 = jnp.sum(acc * acc, axis=0, keepdims=True)
    st_ref[0] = jnp.concatenate([s0, s1], axis=0)


def _conv_stats(xpad_bf16, w9_bf16):
    n, hp, wp, c = xpad_bf16.shape
    h, w = hp - 2, wp - 2
    return pl.pallas_call(
        functools.partial(_conv_stats_kernel, H=h, W=w, C=c),
        out_shape=[
            jax.ShapeDtypeStruct((n, h, w, c), jnp.bfloat16),
            jax.ShapeDtypeStruct((n, 2, c), jnp.float32),
        ],
        grid_spec=pltpu.PrefetchScalarGridSpec(
            num_scalar_prefetch=0,
            grid=(n,),
            in_specs=[
                pl.BlockSpec((1, hp, wp, c), lambda i: (i, 0, 0, 0)),
                pl.BlockSpec((9, c, c), lambda i: (0, 0, 0)),
            ],
            out_specs=[
                pl.BlockSpec((1, h, w, c), lambda i: (i, 0, 0, 0)),
                pl.BlockSpec((1, 2, c), lambda i: (i, 0, 0)),
            ],
        ),
        compiler_params=pltpu.CompilerParams(
            dimension_semantics=("arbitrary",)),
    )(xpad_bf16, w9_bf16)


def _norm_resize_add_kernel(y_ref, st_ref, gb_ref, rh_ref, rwt_ref, res_ref,
                            o_ref, *, inv_m, Hs, Ws, Hd, Wd, C):
    s = jnp.sum(st_ref[...], axis=0)
    mean = s[0:1] * inv_m
    var = jnp.maximum(s[1:2] * inv_m - mean * mean, 0.0)
    scale = gb_ref[0:1] * jax.lax.rsqrt(var + EPS)
    shift = gb_ref[1:2] - mean * scale

    z = jnp.maximum(y_ref[0].astype(jnp.float32) * scale + shift, 0.0)
    t = jnp.dot(rh_ref[...], z.reshape(Hs, Ws * C),
                preferred_element_type=jnp.float32)
    tt = jnp.swapaxes(t.reshape(Hd, Ws, C), 1, 2)
    u = jnp.dot(tt.reshape(Hd * C, Ws), rwt_ref[...],
                preferred_element_type=jnp.float32)
    o_ref[0] = (u.reshape(Hd, C, Wd)
                + res_ref[0].astype(jnp.float32)).astype(o_ref.dtype)


def _norm_resize_add(y_raw, stats, gb, rh, rwt, res_nhcw):
    n, hs, ws, c = y_raw.shape
    hd, wd = rh.shape[0], rwt.shape[1]
    return pl.pallas_call(
        functools.partial(_norm_resize_add_kernel,
                          inv_m=1.0 / float(n * hs * ws),
                          Hs=hs, Ws=ws, Hd=hd, Wd=wd, C=c),
        out_shape=jax.ShapeDtypeStruct((n, hd, c, wd), jnp.bfloat16),
        grid_spec=pltpu.PrefetchScalarGridSpec(
            num_scalar_prefetch=0,
            grid=(n,),
            in_specs=[
                pl.BlockSpec((1, hs, ws, c), lambda i: (i, 0, 0, 0)),
                pl.BlockSpec((n, 2, c), lambda i: (0, 0, 0)),
                pl.BlockSpec((2, c), lambda i: (0, 0)),
                pl.BlockSpec((hd, hs), lambda i: (0, 0)),
                pl.BlockSpec((ws, wd), lambda i: (0, 0)),
                pl.BlockSpec((1, hd, c, wd), lambda i: (i, 0, 0, 0)),
            ],
            out_specs=pl.BlockSpec((1, hd, c, wd), lambda i: (i, 0, 0, 0)),
        ),
        compiler_params=pltpu.CompilerParams(
            dimension_semantics=("arbitrary",)),
    )(y_raw, stats, gb, rh, rwt, res_nhcw)


def kernel(x1, x2, w, b, gamma, beta):
    del b
    c = x1.shape[1]
    h1, w1 = x1.shape[2], x1.shape[3]
    h2, w2 = x2.shape[2], x2.shape[3]

    pad = ((0, 0), (1, 1), (1, 1), (0, 0))
    x1p = jnp.pad(jnp.transpose(x1, (0, 2, 3, 1)), pad).astype(jnp.bfloat16)
    x2p = jnp.pad(jnp.transpose(x2, (0, 2, 3, 1)), pad).astype(jnp.bfloat16)

    w9 = jnp.transpose(w, (2, 3, 1, 0)).reshape(9, c, c).astype(jnp.bfloat16)
    gb = jnp.stack([gamma, beta], axis=0)

    yA, stA = _conv_stats(x2p, w9)
    yB, stB = _conv_stats(x1p, w9)

    res1 = jnp.transpose(x1, (0, 2, 1, 3)).astype(jnp.bfloat16)
    res2 = jnp.transpose(x2, (0, 2, 1, 3)).astype(jnp.bfloat16)

    outA = _norm_resize_add(yA, stA, gb, _interp_matrix(h2, h1),
                            _interp_matrix(w2, w1).T, res1)
    outB = _norm_resize_add(yB, stB, gb, _interp_matrix(h1, h2),
                            _interp_matrix(w1, w2).T, res2)

    x_out = jnp.transpose(outA, (0, 2, 1, 3)).astype(jnp.float32)
    y_out = jnp.transpose(outB, (0, 2, 1, 3)).astype(jnp.float32)
    return x_out, y_out


# speedup vs baseline: 1.3003x; 1.2418x over previous
import functools

import numpy as np

import jax
import jax.numpy as jnp
from jax.experimental import pallas as pl
from jax.experimental.pallas import tpu as pltpu

EPS = 1e-5


def _interp_matrix(d_in, d_out):
    if d_out == 1:
        src = np.zeros((1,), np.float64)
    else:
        src = np.arange(d_out, dtype=np.float64) * ((d_in - 1) / (d_out - 1))
    lo = np.clip(np.floor(src).astype(np.int64), 0, d_in - 1)
    hi = np.minimum(lo + 1, d_in - 1)
    w_hi = (src - lo).astype(np.float32)
    m = np.zeros((d_out, d_in), np.float32)
    m[np.arange(d_out), lo] += 1.0 - w_hi
    m[np.arange(d_out), hi] += w_hi
    return jnp.asarray(m)


def _conv_stats_kernel(xp_ref, w_ref, y_ref, st_ref, *, H, W, C):
    acc = None
    for dy in range(3):
        for dx in range(3):
            xs = xp_ref[0, dy:dy + H, dx:dx + W, :].reshape(H * W, C)
            p = jnp.dot(xs, w_ref[3 * dy + dx],
                        preferred_element_type=jnp.float32)
            acc = p if acc is None else acc + p
    y_ref[0] = acc.reshape(H, W, C).astype(y_ref.dtype)
    s0 = jnp.sum(acc, axis=0, keepdims=True)
    s1 = jnp.sum(acc * acc, axis=0, keepdims=True)
    st_ref[0] = jnp.concatenate([s0, s1], axis=0)


def _conv_stats(xpad_bf16, w9_bf16):
    n, hp, wp, c = xpad_bf16.shape
    h, w = hp - 2, wp - 2
    return pl.pallas_call(
        functools.partial(_conv_stats_kernel, H=h, W=w, C=c),
        out_shape=[
            jax.ShapeDtypeStruct((n, h, w, c), jnp.bfloat16),
            jax.ShapeDtypeStruct((n, 2, c), jnp.float32),
        ],
        grid_spec=pltpu.PrefetchScalarGridSpec(
            num_scalar_prefetch=0,
            grid=(n,),
            in_specs=[
                pl.BlockSpec((1, hp, wp, c), lambda i: (i, 0, 0, 0)),
                pl.BlockSpec((9, c, c), lambda i: (0, 0, 0)),
            ],
            out_specs=[
                pl.BlockSpec((1, h, w, c), lambda i: (i, 0, 0, 0)),
                pl.BlockSpec((1, 2, c), lambda i: (i, 0, 0)),
            ],
        ),
        compiler_params=pltpu.CompilerParams(
            dimension_semantics=("arbitrary",)),
    )(xpad_bf16, w9_bf16)


def _norm_resize_add_kernel(y_ref, st_ref, gb_ref, rh_ref, rwt_ref, res_ref,
                            o_ref, *, inv_m, Hs, Ws, Hd, Wd, C):
    s = jnp.sum(st_ref[...], axis=0)
    mean = s[0:1] * inv_m
    var = jnp.maximum(s[1:2] * inv_m - mean * mean, 0.0)
    scale = gb_ref[0:1] * jax.lax.rsqrt(var + EPS)
    shift = gb_ref[1:2] - mean * scale

    z = jnp.maximum(y_ref[0].astype(jnp.float32) * scale + shift, 0.0)
    t = jnp.dot(rh_ref[...], z.reshape(Hs, Ws * C),
                preferred_element_type=jnp.float32)
    tt = jnp.swapaxes(t.reshape(Hd, Ws, C), 1, 2)
    u = jnp.dot(tt.reshape(Hd * C, Ws), rwt_ref[...],
                preferred_element_type=jnp.float32)
    o_ref[0] = u.reshape(Hd, C, Wd) + res_ref[0]


def _norm_resize_add(y_raw, stats, gb, rh, rwt, res_nhcw):
    n, hs, ws, c = y_raw.shape
    hd, wd = rh.shape[0], rwt.shape[1]
    return pl.pallas_call(
        functools.partial(_norm_resize_add_kernel,
                          inv_m=1.0 / float(n * hs * ws),
                          Hs=hs, Ws=ws, Hd=hd, Wd=wd, C=c),
        out_shape=jax.ShapeDtypeStruct((n, hd, c, wd), jnp.float32),
        grid_spec=pltpu.PrefetchScalarGridSpec(
            num_scalar_prefetch=0,
            grid=(n,),
            in_specs=[
                pl.BlockSpec((1, hs, ws, c), lambda i: (i, 0, 0, 0)),
                pl.BlockSpec((n, 2, c), lambda i: (0, 0, 0)),
                pl.BlockSpec((2, c), lambda i: (0, 0)),
                pl.BlockSpec((hd, hs), lambda i: (0, 0)),
                pl.BlockSpec((ws, wd), lambda i: (0, 0)),
                pl.BlockSpec((1, hd, c, wd), lambda i: (i, 0, 0, 0)),
            ],
            out_specs=pl.BlockSpec((1, hd, c, wd), lambda i: (i, 0, 0, 0)),
        ),
        compiler_params=pltpu.CompilerParams(
            dimension_semantics=("arbitrary",)),
    )(y_raw, stats, gb, rh, rwt, res_nhcw)


def kernel(x1, x2, w, b, gamma, beta):
    del b
    c = x1.shape[1]
    h1, w1 = x1.shape[2], x1.shape[3]
    h2, w2 = x2.shape[2], x2.shape[3]

    pad = ((0, 0), (1, 1), (1, 1), (0, 0))
    x1p = jnp.pad(jnp.transpose(x1, (0, 2, 3, 1)), pad).astype(jnp.bfloat16)
    x2p = jnp.pad(jnp.transpose(x2, (0, 2, 3, 1)), pad).astype(jnp.bfloat16)

    w9 = jnp.transpose(w, (2, 3, 1, 0)).reshape(9, c, c).astype(jnp.bfloat16)
    gb = jnp.stack([gamma, beta], axis=0)

    yA, stA = _conv_stats(x2p, w9)
    yB, stB = _conv_stats(x1p, w9)

    res1 = jnp.transpose(x1, (0, 2, 1, 3))
    res2 = jnp.transpose(x2, (0, 2, 1, 3))

    outA = _norm_resize_add(yA, stA, gb, _interp_matrix(h2, h1),
                            _interp_matrix(w2, w1).T, res1)
    outB = _norm_resize_add(yB, stB, gb, _interp_matrix(h1, h2),
                            _interp_matrix(w1, w2).T, res2)

    x_out = jnp.transpose(outA, (0, 2, 1, 3))
    y_out = jnp.transpose(outB, (0, 2, 1, 3))
    return x_out, y_out
